# Initial kernel scaffold; baseline (speedup 1.0000x reference)
import jax
import jax.numpy as jnp
from jax.experimental import pallas as pl


def kernel(edge_lists, node_states, W, b):
    raise NotImplementedError("write your pallas kernel here")


# speedup vs baseline: 3.0135x; 3.0135x over previous
import functools

import jax
import jax.numpy as jnp
from jax import lax
from jax.experimental import pallas as pl
from jax.experimental.pallas import tpu as pltpu
from jax.experimental.pallas import tpu_sc as plsc

N_NODES = 10000
DIM = 128
NC, NS = 2, 16
NW = NC * NS
CHUNK = 128
DW = 144
N_PAD = 10112
RPT = N_PAD // NS


def _transform_body(x_ref, w_ref, b_ref, o_ref):
    prop = lax.dot_general(
        x_ref[...], w_ref[...],
        dimension_numbers=(((1,), (1,)), ((), ())),
        preferred_element_type=jnp.float32,
    ) + b_ref[...]
    o_ref[:, :DIM] = prop
    col = lax.broadcasted_iota(jnp.int32, (x_ref.shape[0], DW - DIM), 1)
    o_ref[:, DIM:] = jnp.where(col == 0, jnp.float32(1.0), jnp.float32(0.0))


def _transform(node_states, w, b):
    n = node_states.shape[0]
    blk = 2000
    grid = n // blk
    return pl.pallas_call(
        _transform_body,
        grid=(grid,),
        in_specs=[
            pl.BlockSpec((blk, DIM), lambda i: (i, 0)),
            pl.BlockSpec((DIM, DIM), lambda i: (0, 0)),
            pl.BlockSpec((1, DIM), lambda i: (0, 0)),
        ],
        out_specs=pl.BlockSpec((blk, DW), lambda i: (i, 0)),
        out_shape=jax.ShapeDtypeStruct((n, DW), jnp.float32),
    )(node_states, w, b.reshape(1, DIM))


def _make_scatter(cpw):
    mesh = plsc.VectorSubcoreMesh(core_axis_name="c", subcore_axis_name="s",
                                  num_cores=NC, num_subcores=NS)

    @functools.partial(
        pl.kernel,
        out_type=jax.ShapeDtypeStruct((NC, N_PAD, DW), jnp.float32),
        mesh=mesh,
        scratch_types=[
            pltpu.VMEM((cpw, CHUNK), jnp.int32),
            pltpu.VMEM((cpw, CHUNK), jnp.int32),
            pltpu.VMEM((CHUNK, DW), jnp.float32),
            pltpu.VMEM_SHARED((N_PAD, DW), jnp.float32),
            pltpu.SemaphoreType.DMA,
        ],
        compiler_params=pltpu.CompilerParams(use_tc_tiling_on_sc=False),
    )
    def scatter_kernel(src_hbm, tgt_hbm, prop_hbm, out_hbm,
                       src_v, tgt_v, rows_v, acc, sem):
        c = lax.axis_index("c")
        s = lax.axis_index("s")
        wid = s * NC + c

        def zrow(r, carry):
            def zcol(cc, carry2):
                rows_v[r, pl.ds(cc * 16, 16)] = jnp.zeros((16,), jnp.float32)
                return carry2
            return lax.fori_loop(0, DW // 16, zcol, carry)
        lax.fori_loop(0, CHUNK, zrow, 0)

        base = s * RPT
        nfull = RPT // CHUNK
        for k in range(nfull):
            pltpu.sync_copy(rows_v, acc.at[pl.ds(base + k * CHUNK, CHUNK)])
        rem = RPT - nfull * CHUNK
        if rem:
            pltpu.sync_copy(rows_v.at[pl.ds(0, rem)],
                            acc.at[pl.ds(base + nfull * CHUNK, rem)])
        plsc.subcore_barrier()

        pltpu.sync_copy(src_hbm.at[pl.ds(wid * cpw, cpw)], src_v)
        pltpu.sync_copy(tgt_hbm.at[pl.ds(wid * cpw, cpw)], tgt_v)

        def body(j, carry):
            pltpu.async_copy(prop_hbm.at[src_v.at[j]], rows_v, sem).wait()
            pltpu.sync_copy(rows_v, acc.at[tgt_v.at[j]], add=True)
            return carry
        lax.fori_loop(0, cpw, body, 0)

        plsc.subcore_barrier()
        pltpu.sync_copy(acc.at[pl.ds(base, RPT)],
                        out_hbm.at[c, pl.ds(base, RPT)])

    return scatter_kernel


def _finalize_body(p_ref, o_ref):
    p0 = p_ref[0]
    p1 = p_ref[1]
    msgs = p0[:, :DIM] + p1[:, :DIM]
    cnt = p0[:, DIM:DIM + 1] + p1[:, DIM:DIM + 1]
    div = jnp.where(cnt <= jnp.float32(0.0), jnp.float32(1.0), cnt)
    o_ref[...] = msgs / div


def _finalize(parts, n):
    blk = 2000
    grid = n // blk
    return pl.pallas_call(
        _finalize_body,
        grid=(grid,),
        in_specs=[pl.BlockSpec((NC, blk, DW), lambda i: (0, i, 0))],
        out_specs=pl.BlockSpec((blk, DIM), lambda i: (i, 0)),
        out_shape=jax.ShapeDtypeStruct((n, DIM), jnp.float32),
    )(parts)


def kernel(edge_lists, node_states, W, b):
    n = node_states.shape[0]
    e = edge_lists.shape[1]

    cpw = -(-e // (NW * CHUNK))
    cpw = -(-cpw // 8) * 8
    e_pad = NW * cpw * CHUNK
    src = edge_lists[0, :, 0].astype(jnp.int32)
    tgt = edge_lists[0, :, 1].astype(jnp.int32)
    pad = e_pad - e
    src = jnp.concatenate([src, jnp.zeros((pad,), jnp.int32)])
    tgt = jnp.concatenate([tgt, jnp.full((pad,), N_NODES, jnp.int32)])
    src2d = src.reshape(e_pad // CHUNK, CHUNK)
    tgt2d = tgt.reshape(e_pad // CHUNK, CHUNK)

    prop = _transform(node_states, W, b)
    parts = _make_scatter(cpw)(src2d, tgt2d, prop)
    return _finalize(parts, n)


# speedup vs baseline: 3.3774x; 1.1207x over previous
import functools

import jax
import jax.numpy as jnp
from jax import lax
from jax.experimental import pallas as pl
from jax.experimental.pallas import tpu as pltpu
from jax.experimental.pallas import tpu_sc as plsc

N_NODES = 10000
DIM = 128
NC, NS = 2, 16
NW = NC * NS
CHUNK = 64
DW = 144
N_PAD = 10112
RPT = N_PAD // NS


def _transform_body(x_ref, w_ref, b_ref, o_ref):
    prop = lax.dot_general(
        x_ref[...], w_ref[...],
        dimension_numbers=(((1,), (1,)), ((), ())),
        preferred_element_type=jnp.float32,
    ) + b_ref[...]
    o_ref[:, :DIM] = prop
    col = lax.broadcasted_iota(jnp.int32, (x_ref.shape[0], DW - DIM), 1)
    o_ref[:, DIM:] = jnp.where(col == 0, jnp.float32(1.0), jnp.float32(0.0))


def _transform(node_states, w, b):
    n = node_states.shape[0]
    blk = 2000
    grid = n // blk
    return pl.pallas_call(
        _transform_body,
        grid=(grid,),
        in_specs=[
            pl.BlockSpec((blk, DIM), lambda i: (i, 0)),
            pl.BlockSpec((DIM, DIM), lambda i: (0, 0)),
            pl.BlockSpec((1, DIM), lambda i: (0, 0)),
        ],
        out_specs=pl.BlockSpec((blk, DW), lambda i: (i, 0)),
        out_shape=jax.ShapeDtypeStruct((n, DW), jnp.float32),
    )(node_states, w, b.reshape(1, DIM))


def _make_scatter(cpw):
    mesh = plsc.VectorSubcoreMesh(core_axis_name="c", subcore_axis_name="s",
                                  num_cores=NC, num_subcores=NS)

    @functools.partial(
        pl.kernel,
        out_type=jax.ShapeDtypeStruct((NC, N_PAD, DW), jnp.float32),
        mesh=mesh,
        scratch_types=[
            pltpu.VMEM((cpw, CHUNK), jnp.int32),
            pltpu.VMEM((cpw, CHUNK), jnp.int32),
            pltpu.VMEM((CHUNK, DW), jnp.float32),
            pltpu.VMEM((CHUNK, DW), jnp.float32),
            pltpu.VMEM_SHARED((N_PAD, DW), jnp.float32),
            pltpu.SemaphoreType.DMA,
            pltpu.SemaphoreType.DMA,
        ],
        compiler_params=pltpu.CompilerParams(use_tc_tiling_on_sc=False),
    )
    def scatter_kernel(src_hbm, tgt_hbm, prop_hbm, out_hbm,
                       src_v, tgt_v, rows_a, rows_b, acc, sem_a, sem_b):
        rows_v = rows_a
        c = lax.axis_index("c")
        s = lax.axis_index("s")
        wid = s * NC + c

        def zrow(r, carry):
            def zcol(cc, carry2):
                rows_v[r, pl.ds(cc * 16, 16)] = jnp.zeros((16,), jnp.float32)
                return carry2
            return lax.fori_loop(0, DW // 16, zcol, carry)
        lax.fori_loop(0, CHUNK, zrow, 0)

        base = s * RPT
        nfull = RPT // CHUNK
        for k in range(nfull):
            pltpu.sync_copy(rows_v, acc.at[pl.ds(base + k * CHUNK, CHUNK)])
        rem = RPT - nfull * CHUNK
        if rem:
            pltpu.sync_copy(rows_v.at[pl.ds(0, rem)],
                            acc.at[pl.ds(base + nfull * CHUNK, rem)])
        plsc.subcore_barrier()

        pltpu.sync_copy(src_hbm.at[pl.ds(wid * cpw, cpw)], src_v)
        pltpu.sync_copy(tgt_hbm.at[pl.ds(wid * cpw, cpw)], tgt_v)

        pltpu.async_copy(prop_hbm.at[src_v.at[0]], rows_a, sem_a)

        def body(jj, carry):
            j = jj * 2
            pltpu.async_copy(prop_hbm.at[src_v.at[j + 1]], rows_b, sem_b)
            pltpu.make_async_copy(prop_hbm.at[src_v.at[j]], rows_a, sem_a).wait()
            pltpu.sync_copy(rows_a, acc.at[tgt_v.at[j]], add=True)

            @pl.when(j + 2 < cpw)
            def _():
                pltpu.async_copy(prop_hbm.at[src_v.at[j + 2]], rows_a, sem_a)
            pltpu.make_async_copy(prop_hbm.at[src_v.at[j + 1]], rows_b,
                                  sem_b).wait()
            pltpu.sync_copy(rows_b, acc.at[tgt_v.at[j + 1]], add=True)
            return carry
        lax.fori_loop(0, cpw // 2, body, 0)

        plsc.subcore_barrier()
        pltpu.sync_copy(acc.at[pl.ds(base, RPT)],
                        out_hbm.at[c, pl.ds(base, RPT)])

    return scatter_kernel


def _finalize_body(p_ref, o_ref):
    p0 = p_ref[0]
    p1 = p_ref[1]
    msgs = p0[:, :DIM] + p1[:, :DIM]
    cnt = p0[:, DIM:DIM + 1] + p1[:, DIM:DIM + 1]
    div = jnp.where(cnt <= jnp.float32(0.0), jnp.float32(1.0), cnt)
    o_ref[...] = msgs / div


def _finalize(parts, n):
    blk = 2000
    grid = n // blk
    return pl.pallas_call(
        _finalize_body,
        grid=(grid,),
        in_specs=[pl.BlockSpec((NC, blk, DW), lambda i: (0, i, 0))],
        out_specs=pl.BlockSpec((blk, DIM), lambda i: (i, 0)),
        out_shape=jax.ShapeDtypeStruct((n, DIM), jnp.float32),
    )(parts)


def kernel(edge_lists, node_states, W, b):
    n = node_states.shape[0]
    e = edge_lists.shape[1]

    cpw = -(-e // (NW * CHUNK))
    cpw = -(-cpw // 8) * 8
    e_pad = NW * cpw * CHUNK
    src = edge_lists[0, :, 0].astype(jnp.int32)
    tgt = edge_lists[0, :, 1].astype(jnp.int32)
    pad = e_pad - e
    src = jnp.concatenate([src, jnp.zeros((pad,), jnp.int32)])
    tgt = jnp.concatenate([tgt, jnp.full((pad,), N_NODES, jnp.int32)])
    src2d = src.reshape(e_pad // CHUNK, CHUNK)
    tgt2d = tgt.reshape(e_pad // CHUNK, CHUNK)

    prop = _transform(node_states, W, b)
    parts = _make_scatter(cpw)(src2d, tgt2d, prop)
    return _finalize(parts, n)


# speedup vs baseline: 7.0556x; 2.0891x over previous
import functools

import jax
import jax.numpy as jnp
from jax import lax
from jax.experimental import pallas as pl
from jax.experimental.pallas import tpu as pltpu
from jax.experimental.pallas import tpu_sc as plsc

N_NODES = 10000
DIM = 128
HALF = DIM // 2
NC, NS = 2, 16
CHUNK = 64
DW = 80
N_PAD = 10112
RPT = N_PAD // NS
PASS_CHUNKS = 40


def _transform_body(x_ref, w_ref, b_ref, o_ref):
    blk = x_ref.shape[0]
    prop = lax.dot_general(
        x_ref[...], w_ref[...],
        dimension_numbers=(((1,), (1,)), ((), ())),
        preferred_element_type=jnp.float32,
    ) + b_ref[...]
    col = lax.broadcasted_iota(jnp.int32, (blk, DW - HALF), 1)
    cnt_pad = jnp.where(col == 0, jnp.float32(1.0), jnp.float32(0.0))
    o_ref[0] = jnp.concatenate([prop[:, :HALF], cnt_pad], axis=1)
    o_ref[1] = jnp.concatenate([prop[:, HALF:], cnt_pad * 0.0], axis=1)


def _transform(node_states, w, b):
    blk = 1264
    grid = N_PAD // blk
    return pl.pallas_call(
        _transform_body,
        grid=(grid,),
        in_specs=[
            pl.BlockSpec((blk, DIM), lambda i: (i, 0)),
            pl.BlockSpec((DIM, DIM), lambda i: (0, 0)),
            pl.BlockSpec((1, DIM), lambda i: (0, 0)),
        ],
        out_specs=pl.BlockSpec((NC, blk, DW), lambda i: (0, i, 0)),
        out_shape=jax.ShapeDtypeStruct((NC, N_PAD, DW), jnp.float32),
    )(node_states, w, b.reshape(1, DIM))


def _make_scatter(cpt):
    mesh = plsc.VectorSubcoreMesh(core_axis_name="c", subcore_axis_name="s",
                                  num_cores=NC, num_subcores=NS)
    npass = cpt // PASS_CHUNKS

    @functools.partial(
        pl.kernel,
        out_type=jax.ShapeDtypeStruct((NC, N_PAD, DW), jnp.float32),
        mesh=mesh,
        scratch_types=[
            pltpu.VMEM((PASS_CHUNKS, 2, CHUNK), jnp.int32),
            pltpu.VMEM((CHUNK, DW), jnp.float32),
            pltpu.VMEM((CHUNK, DW), jnp.float32),
            pltpu.VMEM_SHARED((N_PAD, DW), jnp.float32),
            pltpu.VMEM_SHARED((N_PAD, DW), jnp.float32),
            pltpu.SemaphoreType.DMA,
            pltpu.SemaphoreType.DMA,
        ],
        compiler_params=pltpu.CompilerParams(use_tc_tiling_on_sc=False),
    )
    def scatter_kernel(st_hbm, prop_hbm, out_hbm,
                       idx_blk, rows_a, rows_b, tab, acc, sem_a, sem_b):
        c = lax.axis_index("c")
        s = lax.axis_index("s")
        base = s * RPT

        def zrow(r, carry):
            def zcol(cc, carry2):
                rows_a[r, pl.ds(cc * 16, 16)] = jnp.zeros((16,), jnp.float32)
                return carry2
            return lax.fori_loop(0, DW // 16, zcol, carry)
        lax.fori_loop(0, CHUNK, zrow, 0)

        nfull = RPT // CHUNK
        for k in range(nfull):
            pltpu.sync_copy(rows_a, acc.at[pl.ds(base + k * CHUNK, CHUNK)])
        rem = RPT - nfull * CHUNK
        if rem:
            pltpu.sync_copy(rows_a.at[pl.ds(0, rem)],
                            acc.at[pl.ds(base + nfull * CHUNK, rem)])

        pltpu.sync_copy(prop_hbm.at[c, pl.ds(base, RPT)],
                        tab.at[pl.ds(base, RPT)])
        plsc.subcore_barrier()

        def pass_body(p, carry):
            pltpu.sync_copy(
                st_hbm.at[pl.ds(s * cpt + p * PASS_CHUNKS, PASS_CHUNKS)],
                idx_blk)
            pltpu.async_copy(tab.at[idx_blk.at[0, 0]], rows_a, sem_a)

            def body(kk, c2):
                k = kk * 2
                pltpu.async_copy(tab.at[idx_blk.at[k + 1, 0]], rows_b, sem_b)
                pltpu.make_async_copy(tab.at[idx_blk.at[k, 0]], rows_a,
                                      sem_a).wait()
                pltpu.sync_copy(rows_a, acc.at[idx_blk.at[k, 1]], add=True)

                @pl.when(k + 2 < PASS_CHUNKS)
                def _():
                    pltpu.async_copy(tab.at[idx_blk.at[k + 2, 0]], rows_a,
                                     sem_a)
                pltpu.make_async_copy(tab.at[idx_blk.at[k + 1, 0]], rows_b,
                                      sem_b).wait()
                pltpu.sync_copy(rows_b, acc.at[idx_blk.at[k + 1, 1]],
                                add=True)
                return c2
            lax.fori_loop(0, PASS_CHUNKS // 2, body, 0)
            return carry
        lax.fori_loop(0, npass, pass_body, 0)

        plsc.subcore_barrier()
        pltpu.sync_copy(acc.at[pl.ds(base, RPT)],
                        out_hbm.at[c, pl.ds(base, RPT)])

    return scatter_kernel


def _finalize_body(p_ref, o_ref):
    p0 = p_ref[0]
    p1 = p_ref[1]
    cnt = p0[:, HALF:HALF + 1]
    div = jnp.where(cnt <= jnp.float32(0.0), jnp.float32(1.0), cnt)
    o_ref[...] = jnp.concatenate([p0[:, :HALF], p1[:, :HALF]], axis=1) / div


def _finalize(parts, n):
    blk = 2000
    grid = n // blk
    return pl.pallas_call(
        _finalize_body,
        grid=(grid,),
        in_specs=[pl.BlockSpec((NC, blk, DW), lambda i: (0, i, 0))],
        out_specs=pl.BlockSpec((blk, DIM), lambda i: (i, 0)),
        out_shape=jax.ShapeDtypeStruct((n, DIM), jnp.float32),
    )(parts)


def kernel(edge_lists, node_states, W, b):
    n = node_states.shape[0]
    e = edge_lists.shape[1]

    cpt = -(-e // (NS * CHUNK))
    cpt = -(-cpt // PASS_CHUNKS) * PASS_CHUNKS
    e_pad = NS * cpt * CHUNK
    src = edge_lists[0, :, 0].astype(jnp.int32)
    tgt = edge_lists[0, :, 1].astype(jnp.int32)
    pad = e_pad - e
    src = jnp.concatenate([src, jnp.zeros((pad,), jnp.int32)])
    tgt = jnp.concatenate([tgt, jnp.full((pad,), N_NODES, jnp.int32)])
    st = jnp.stack([src.reshape(e_pad // CHUNK, CHUNK),
                    tgt.reshape(e_pad // CHUNK, CHUNK)], axis=1)

    ns_pad = jnp.concatenate(
        [node_states, jnp.zeros((N_PAD - n, DIM), jnp.float32)])
    prop = _transform(ns_pad, W, b)
    parts = _make_scatter(cpt)(st, prop)
    return _finalize(parts, n)


# speedup vs baseline: 7.2737x; 1.0309x over previous
import functools

import jax
import jax.numpy as jnp
from jax import lax
from jax.experimental import pallas as pl
from jax.experimental.pallas import tpu as pltpu
from jax.experimental.pallas import tpu_sc as plsc

N_NODES = 10000
DIM = 128
HALF = DIM // 2
NC, NS = 2, 16
CHUNK = 64
DW = 80
N_PAD = 10112
RPT = N_PAD // NS
RPT_LAST = N_NODES - (NS - 1) * RPT
PASS_CHUNKS = 40


def _transform_body(x_ref, w_ref, b_ref, o_ref):
    blk = x_ref.shape[0]
    prop = lax.dot_general(
        x_ref[...], w_ref[...],
        dimension_numbers=(((1,), (1,)), ((), ())),
        preferred_element_type=jnp.float32,
    ) + b_ref[...]
    col = lax.broadcasted_iota(jnp.int32, (blk, DW - HALF), 1)
    cnt_pad = jnp.where(col == 0, jnp.float32(1.0), jnp.float32(0.0))
    o_ref[0] = jnp.concatenate([prop[:, :HALF], cnt_pad], axis=1)
    o_ref[1] = jnp.concatenate([prop[:, HALF:], cnt_pad], axis=1)


def _transform(node_states, w, b):
    blk = 1264
    grid = N_PAD // blk
    return pl.pallas_call(
        _transform_body,
        grid=(grid,),
        in_specs=[
            pl.BlockSpec((blk, DIM), lambda i: (i, 0)),
            pl.BlockSpec((DIM, DIM), lambda i: (0, 0)),
            pl.BlockSpec((1, DIM), lambda i: (0, 0)),
        ],
        out_specs=pl.BlockSpec((NC, blk, DW), lambda i: (0, i, 0)),
        out_shape=jax.ShapeDtypeStruct((NC, N_PAD, DW), jnp.float32),
    )(node_states, w, b.reshape(1, DIM))


def _make_scatter(cpt):
    mesh = plsc.VectorSubcoreMesh(core_axis_name="c", subcore_axis_name="s",
                                  num_cores=NC, num_subcores=NS)
    npass = cpt // PASS_CHUNKS

    @functools.partial(
        pl.kernel,
        out_type=jax.ShapeDtypeStruct((N_NODES, DIM), jnp.float32),
        mesh=mesh,
        scratch_types=[
            pltpu.VMEM((PASS_CHUNKS, 2, CHUNK), jnp.int32),
            pltpu.VMEM((CHUNK, DW), jnp.float32),
            pltpu.VMEM((CHUNK, DW), jnp.float32),
            pltpu.VMEM((CHUNK, HALF), jnp.float32),
            pltpu.VMEM_SHARED((N_PAD, DW), jnp.float32),
            pltpu.VMEM_SHARED((N_PAD, DW), jnp.float32),
            pltpu.SemaphoreType.DMA,
            pltpu.SemaphoreType.DMA,
        ],
        compiler_params=pltpu.CompilerParams(use_tc_tiling_on_sc=False),
    )
    def scatter_kernel(st_hbm, prop_hbm, out_hbm,
                       idx_blk, rows_a, rows_b, obuf, tab, acc,
                       sem_a, sem_b):
        c = lax.axis_index("c")
        s = lax.axis_index("s")
        base = s * RPT

        tab_cp = pltpu.async_copy(prop_hbm.at[c, pl.ds(base, RPT)],
                                  tab.at[pl.ds(base, RPT)], sem_b)

        def zrow(r, carry):
            def zcol(cc, carry2):
                rows_a[r, pl.ds(cc * 16, 16)] = jnp.zeros((16,), jnp.float32)
                return carry2
            return lax.fori_loop(0, DW // 16, zcol, carry)
        lax.fori_loop(0, CHUNK, zrow, 0)

        nfull = RPT // CHUNK
        for k in range(nfull):
            pltpu.sync_copy(rows_a, acc.at[pl.ds(base + k * CHUNK, CHUNK)])
        rem = RPT - nfull * CHUNK
        if rem:
            pltpu.sync_copy(rows_a.at[pl.ds(0, rem)],
                            acc.at[pl.ds(base + nfull * CHUNK, rem)])
        tab_cp.wait()
        plsc.subcore_barrier()

        def pass_body(p, carry):
            pltpu.sync_copy(
                st_hbm.at[pl.ds(s * cpt + p * PASS_CHUNKS, PASS_CHUNKS)],
                idx_blk)
            pltpu.async_copy(tab.at[idx_blk.at[0, 0]], rows_a, sem_a)

            def body(kk, c2):
                k = kk * 2
                pltpu.async_copy(tab.at[idx_blk.at[k + 1, 0]], rows_b, sem_b)
                pltpu.make_async_copy(tab.at[idx_blk.at[k, 0]], rows_a,
                                      sem_a).wait()
                pltpu.sync_copy(rows_a, acc.at[idx_blk.at[k, 1]], add=True)

                @pl.when(k + 2 < PASS_CHUNKS)
                def _():
                    pltpu.async_copy(tab.at[idx_blk.at[k + 2, 0]], rows_a,
                                     sem_a)
                pltpu.make_async_copy(tab.at[idx_blk.at[k + 1, 0]], rows_b,
                                      sem_b).wait()
                pltpu.sync_copy(rows_b, acc.at[idx_blk.at[k + 1, 1]],
                                add=True)
                return c2
            lax.fori_loop(0, PASS_CHUNKS // 2, body, 0)
            return carry
        lax.fori_loop(0, npass, pass_body, 0)

        plsc.subcore_barrier()

        one16 = jnp.full((16,), 1.0, jnp.float32)

        def emit_block(r0, nrows):
            pltpu.sync_copy(acc.at[pl.ds(r0, nrows)],
                            rows_a.at[pl.ds(0, nrows)])

            def drow(r, carry):
                cnt = jnp.full((16,), rows_a[r, pl.ds(HALF, 16)][0],
                               jnp.float32)
                inv = one16 / jnp.maximum(cnt, one16)
                for q in range(HALF // 16):
                    obuf[r, pl.ds(q * 16, 16)] = (
                        rows_a[r, pl.ds(q * 16, 16)] * inv)
                return carry
            lax.fori_loop(0, nrows, drow, 0)
            pltpu.sync_copy(obuf.at[pl.ds(0, nrows)],
                            out_hbm.at[pl.ds(r0, nrows),
                                       pl.ds(c * HALF, HALF)])

        def emit_rows(total):
            nb = total // CHUNK
            for k in range(nb):
                emit_block(base + k * CHUNK, CHUNK)
            tail = total - nb * CHUNK
            if tail:
                emit_block(base + nb * CHUNK, tail)

        @pl.when(s < NS - 1)
        def _():
            emit_rows(RPT)

        @pl.when(s == NS - 1)
        def _():
            emit_rows(RPT_LAST)

    return scatter_kernel


def kernel(edge_lists, node_states, W, b):
    e = edge_lists.shape[1]

    cpt = -(-e // (NS * CHUNK))
    cpt = -(-cpt // PASS_CHUNKS) * PASS_CHUNKS
    e_pad = NS * cpt * CHUNK
    src = edge_lists[0, :, 0].astype(jnp.int32)
    tgt = edge_lists[0, :, 1].astype(jnp.int32)
    pad = e_pad - e
    src = jnp.concatenate([src, jnp.zeros((pad,), jnp.int32)])
    tgt = jnp.concatenate([tgt, jnp.full((pad,), N_NODES, jnp.int32)])
    st = jnp.stack([src.reshape(e_pad // CHUNK, CHUNK),
                    tgt.reshape(e_pad // CHUNK, CHUNK)], axis=1)

    prop = _transform(node_states, W, b)
    return _make_scatter(cpt)(st, prop)


# speedup vs baseline: 7.3380x; 1.0088x over previous
import functools

import jax
import jax.numpy as jnp
from jax import lax
from jax.experimental import pallas as pl
from jax.experimental.pallas import tpu as pltpu
from jax.experimental.pallas import tpu_sc as plsc

N_NODES = 10000
DIM = 128
HALF = DIM // 2
NC, NS = 2, 16
CHUNK = 40
DW = 72
N_PAD = 10112
RPT = N_PAD // NS
RPT_LAST = N_NODES - (NS - 1) * RPT
PASS_CHUNKS = 25
PASS_E = PASS_CHUNKS * CHUNK


def _transform_body(x_ref, w_ref, b_ref, o_ref):
    blk = x_ref.shape[0]
    prop = lax.dot_general(
        x_ref[...], w_ref[...],
        dimension_numbers=(((1,), (1,)), ((), ())),
        preferred_element_type=jnp.float32,
    ) + b_ref[...]
    col = lax.broadcasted_iota(jnp.int32, (blk, DW - HALF), 1)
    cnt_pad = jnp.where(col == 0, jnp.float32(1.0), jnp.float32(0.0))
    o_ref[0] = jnp.concatenate([prop[:, :HALF], cnt_pad], axis=1)
    o_ref[1] = jnp.concatenate([prop[:, HALF:], cnt_pad], axis=1)


def _transform(node_states, w, b):
    blk = 1264
    grid = N_PAD // blk
    return pl.pallas_call(
        _transform_body,
        grid=(grid,),
        in_specs=[
            pl.BlockSpec((blk, DIM), lambda i: (i, 0)),
            pl.BlockSpec((DIM, DIM), lambda i: (0, 0)),
            pl.BlockSpec((1, DIM), lambda i: (0, 0)),
        ],
        out_specs=pl.BlockSpec((NC, blk, DW), lambda i: (0, i, 0)),
        out_shape=jax.ShapeDtypeStruct((NC, N_PAD, DW), jnp.float32),
    )(node_states, w, b.reshape(1, DIM))


def _make_scatter(ept):
    mesh = plsc.VectorSubcoreMesh(core_axis_name="c", subcore_axis_name="s",
                                  num_cores=NC, num_subcores=NS)
    npass = ept // PASS_E
    assert npass % 2 == 0 and npass * PASS_E == ept

    @functools.partial(
        pl.kernel,
        out_type=jax.ShapeDtypeStruct((N_NODES, DIM), jnp.float32),
        mesh=mesh,
        scratch_types=[
            pltpu.VMEM((2, PASS_CHUNKS, 2, CHUNK), jnp.int32),
            pltpu.VMEM((CHUNK, DW), jnp.float32),
            pltpu.VMEM((CHUNK, DW), jnp.float32),
            pltpu.VMEM((CHUNK, HALF), jnp.float32),
            pltpu.VMEM_SHARED((N_PAD, DW), jnp.float32),
            pltpu.VMEM_SHARED((N_PAD, DW), jnp.float32),
            pltpu.SemaphoreType.DMA,
            pltpu.SemaphoreType.DMA,
            pltpu.SemaphoreType.DMA,
        ],
        compiler_params=pltpu.CompilerParams(use_tc_tiling_on_sc=False,
                                             needs_layout_passes=False),
    )
    def scatter_kernel(st_hbm, prop_hbm, out_hbm,
                       idx_blk, rows_a, rows_b, obuf, tab, acc,
                       sem_a, sem_b, sem_p):
        c = lax.axis_index("c")
        s = lax.axis_index("s")
        base = s * RPT
        rows_by_par = (rows_a, rows_b)
        sems = (sem_a, sem_b)

        cpt = ept // CHUNK
        tab_cp = pltpu.async_copy(prop_hbm.at[c, pl.ds(base, RPT)],
                                  tab.at[pl.ds(base, RPT)], sem_b)
        pltpu.async_copy(st_hbm.at[pl.ds(s * cpt, PASS_CHUNKS)],
                         idx_blk.at[0], sem_p)

        def zrow(r, carry):
            def zcol(cc, carry2):
                rows_a[r, pl.ds(cc * 16, 16)] = jnp.zeros((16,), jnp.float32)
                return carry2
            return lax.fori_loop(0, DW // 16, zcol, carry)
        lax.fori_loop(0, CHUNK, zrow, 0)

        nfull = RPT // CHUNK
        for k in range(nfull):
            pltpu.sync_copy(rows_a, acc.at[pl.ds(base + k * CHUNK, CHUNK)])
        rem = RPT - nfull * CHUNK
        if rem:
            pltpu.sync_copy(rows_a.at[pl.ds(0, rem)],
                            acc.at[pl.ds(base + nfull * CHUNK, rem)])
        tab_cp.wait()
        pltpu.make_async_copy(st_hbm.at[pl.ds(s * cpt, PASS_CHUNKS)],
                              idx_blk.at[0], sem_p).wait()
        plsc.subcore_barrier()

        def gather(slot, k, par):
            pltpu.async_copy(tab.at[idx_blk.at[slot, k, 0]],
                             rows_by_par[par], sems[par])

        def wait_scatter(slot, k, par):
            pltpu.make_async_copy(tab.at[idx_blk.at[slot, k, 0]],
                                  rows_by_par[par], sems[par]).wait()
            pltpu.sync_copy(rows_by_par[par], acc.at[idx_blk.at[slot, k, 1]],
                            add=True)

        def pass_work(p, slot, other):
            @pl.when(p + 1 < npass)
            def _():
                pltpu.async_copy(
                    st_hbm.at[pl.ds(s * cpt + (p + 1) * PASS_CHUNKS,
                                    PASS_CHUNKS)],
                    idx_blk.at[other], sem_p)

            gather(slot, 0, 0)

            def body(kk, c2):
                k = kk * 2
                gather(slot, k + 1, 1)
                wait_scatter(slot, k, 0)
                gather(slot, k + 2, 0)
                wait_scatter(slot, k + 1, 1)
                return c2
            lax.fori_loop(0, (PASS_CHUNKS - 1) // 2, body, 0)
            wait_scatter(slot, PASS_CHUNKS - 1, 0)

            @pl.when(p + 1 < npass)
            def _():
                pltpu.make_async_copy(
                    st_hbm.at[pl.ds(s * cpt + (p + 1) * PASS_CHUNKS,
                                    PASS_CHUNKS)],
                    idx_blk.at[other], sem_p).wait()

        def outer(q, carry):
            pass_work(q * 2, 0, 1)
            pass_work(q * 2 + 1, 1, 0)
            return carry
        lax.fori_loop(0, npass // 2, outer, 0)

        plsc.subcore_barrier()

        one16 = jnp.full((16,), 1.0, jnp.float32)

        def emit_block(r0, nrows):
            pltpu.sync_copy(acc.at[pl.ds(r0, nrows)],
                            rows_a.at[pl.ds(0, nrows)])

            def drow(r, carry):
                cnt = jnp.full(
                    (16,), rows_a[r, pl.ds(DW - 16, 16)][HALF - (DW - 16)],
                    jnp.float32)
                inv = one16 / jnp.maximum(cnt, one16)
                for q in range(HALF // 16):
                    obuf[r, pl.ds(q * 16, 16)] = (
                        rows_a[r, pl.ds(q * 16, 16)] * inv)
                return carry
            lax.fori_loop(0, nrows, drow, 0)
            pltpu.sync_copy(obuf.at[pl.ds(0, nrows)],
                            out_hbm.at[pl.ds(r0, nrows),
                                       pl.ds(c * HALF, HALF)])

        def emit_rows(total):
            nb = total // CHUNK
            for k in range(nb):
                emit_block(base + k * CHUNK, CHUNK)
            tail = total - nb * CHUNK
            if tail:
                emit_block(base + nb * CHUNK, tail)

        @pl.when(s < NS - 1)
        def _():
            emit_rows(RPT)

        @pl.when(s == NS - 1)
        def _():
            emit_rows(RPT_LAST)

    return scatter_kernel


def kernel(edge_lists, node_states, W, b):
    e = edge_lists.shape[1]
    ept = e // NS
    src = edge_lists[0, :, 0].astype(jnp.int32)
    tgt = edge_lists[0, :, 1].astype(jnp.int32)
    st = jnp.stack([src.reshape(e // CHUNK, CHUNK),
                    tgt.reshape(e // CHUNK, CHUNK)], axis=1)
    prop = _transform(node_states, W, b)
    return _make_scatter(ept)(st, prop)
